# baseline (speedup 1.0000x reference)
import functools

import jax
import jax.numpy as jnp
import numpy as np
from jax import lax
from jax.experimental import pallas as pl
from jax.experimental.pallas import tpu as pltpu
from jax.experimental.pallas import tpu_sc as plsc

_BATCH = 16384
_NODES = 1024
_PANEL = 128

_SC_ROWS = 2048
_SC_BASE = _BATCH - _SC_ROWS
_TB = 2048
_INFO = plsc.get_sparse_core_info()
_NC, _NS = _INFO.num_cores, _INFO.num_subcores
_NW = _NC * _NS
_ROWS_PER_W = _SC_ROWS // _NW
_CH = 32
_NCHUNK = _ROWS_PER_W // _CH
_CHW = _CH * _NODES


def _ancestor_maps():
    parent = np.arange(_PANEL)
    parent[1:31] = (np.arange(1, 31) - 1) // 2
    maps = []
    anc = np.arange(_PANEL)
    for _ in range(4):
        anc = parent[anc]
        m = np.zeros((_PANEL, _PANEL), dtype=np.float32)
        m[anc, np.arange(_PANEL)] = 1.0
        maps.append(m)
    return np.stack(maps)


_ANC_MAPS = _ancestor_maps()


def _tc_body(x_ref, sel_ref, o_ref):
    xb = x_ref[...]
    head = xb[:, :_PANEL]
    m = head
    sel = sel_ref[...]
    for k in range(4):
        m = jnp.minimum(
            m, jax.lax.dot(head, sel[k], preferred_element_type=jnp.float32))
    fixed = jnp.concatenate([m, xb[:, _PANEL:]], axis=1)
    o_ref[...] = 1.0 / (1.0 + jnp.exp(-fixed))


def _tc_part(x):
    grid = (_BATCH - _SC_ROWS) // _TB
    return pl.pallas_call(
        _tc_body,
        grid=(grid,),
        in_specs=[
            pl.BlockSpec((_TB, _NODES), lambda i: (i, 0)),
            pl.BlockSpec((4, _PANEL, _PANEL), lambda i: (0, 0, 0)),
        ],
        out_specs=pl.BlockSpec((_TB, _NODES), lambda i: (i, 0)),
        out_shape=jax.ShapeDtypeStruct((_BATCH, _NODES), jnp.float32),
        compiler_params=pltpu.CompilerParams(
            dimension_semantics=("parallel",),
        ),
    )(x, jnp.asarray(_ANC_MAPS))


def _vgather(v, idx):
    return lax.gather(
        v,
        idx[:, None],
        dimension_numbers=lax.GatherDimensionNumbers(
            offset_dims=(), collapsed_slice_dims=(0,), start_index_map=(0,)),
        slice_sizes=(1,),
        mode=lax.GatherScatterMode.PROMISE_IN_BOUNDS,
    )


def _tree_fix(buf, r, p1, p2, p4, p_hi, i16):
    lo = buf[r, pl.ds(0, 16)]
    lo = jnp.minimum(lo, _vgather(lo, p1))
    lo = jnp.minimum(lo, _vgather(lo, p2))
    lo = jnp.minimum(lo, _vgather(lo, p4))
    buf[r, pl.ds(0, 16)] = lo
    hi = buf[r, pl.ds(16, 16)]
    gh = _vgather(lo, p_hi)
    hi = jnp.where(i16 < 15, jnp.minimum(hi, gh), hi)
    buf[r, pl.ds(16, 16)] = hi


def _compute_chunk(buf, p1, p2, p4, p_hi, i16):
    def row(r, c):
        _tree_fix(buf, r, p1, p2, p4, p_hi, i16)
        vals = [buf[r, pl.ds(j * 16, 16)] for j in range(64)]
        outs = [1.0 / (1.0 + jnp.exp(-v)) for v in vals]
        for j in range(64):
            buf[r, pl.ds(j * 16, 16)] = outs[j]
        return c

    lax.fori_loop(0, _CH, row, 0)


def _sc_body(x_hbm, o_hbm, b0, b1, ls0, ls1, ss0, ss1):
    wid = lax.axis_index("s") * _NC + lax.axis_index("c")
    r0 = wid * _ROWS_PER_W
    rx = _SC_BASE + r0

    i16 = lax.iota(jnp.int32, 16)
    par = lambda v: jnp.maximum((v - 1) >> 1, 0)
    p1 = par(i16)
    p2 = par(p1)
    p4 = par(par(p2))
    p_hi = (i16 + 15) >> 1

    bufs = (b0, b1)
    lsems = (ls0, ls1)
    ssems = (ss0, ss1)

    def ld(k):
        return pltpu.make_async_copy(
            x_hbm.at[pl.ds(rx + k * _CH, _CH), :], bufs[k % 2], lsems[k % 2])

    def st(k):
        return pltpu.make_async_copy(
            bufs[k % 2], o_hbm.at[pl.ds(r0 + k * _CH, _CH), :], ssems[k % 2])

    ld(0).start()
    for k in range(_NCHUNK):
        if k + 1 < _NCHUNK:
            if k - 1 >= 0:
                st(k - 1).wait()
            ld(k + 1).start()
        ld(k).wait()
        _compute_chunk(bufs[k % 2], p1, p2, p4, p_hi, i16)
        st(k).start()
    if _NCHUNK >= 2:
        st(_NCHUNK - 2).wait()
    st(_NCHUNK - 1).wait()


def _sc_part(x):
    mesh = plsc.VectorSubcoreMesh(core_axis_name="c", subcore_axis_name="s")
    run = functools.partial(
        pl.kernel,
        mesh=mesh,
        out_type=jax.ShapeDtypeStruct((_SC_ROWS, _NODES), jnp.float32),
        scratch_types=[
            pltpu.VMEM((_CH, _NODES), jnp.float32),
            pltpu.VMEM((_CH, _NODES), jnp.float32),
            pltpu.SemaphoreType.DMA,
            pltpu.SemaphoreType.DMA,
            pltpu.SemaphoreType.DMA,
            pltpu.SemaphoreType.DMA,
        ],
        compiler_params=pltpu.CompilerParams(use_tc_tiling_on_sc=True),
    )(_sc_body)
    return run(x)


@jax.jit
def kernel(x):
    sc_out = _sc_part(x)
    tc_out = _tc_part(x)
    return lax.dynamic_update_slice(tc_out, sc_out, (_SC_BASE, 0))


# speedup vs baseline: 1.1280x; 1.1280x over previous
import functools

import jax
import jax.numpy as jnp
import numpy as np
from jax import lax
from jax.experimental import pallas as pl
from jax.experimental.pallas import tpu as pltpu
from jax.experimental.pallas import tpu_sc as plsc

_BATCH = 16384
_NODES = 1024
_PANEL = 128

_SC_ROWS = 1024
_SC_BASE = _BATCH - _SC_ROWS
_TB = 1920
_INFO = plsc.get_sparse_core_info()
_NC, _NS = _INFO.num_cores, _INFO.num_subcores
_NW = _NC * _NS
_ROWS_PER_W = _SC_ROWS // _NW
_CH = 32
_NCHUNK = _ROWS_PER_W // _CH
_CHW = _CH * _NODES


def _ancestor_maps():
    parent = np.arange(_PANEL)
    parent[1:31] = (np.arange(1, 31) - 1) // 2
    maps = []
    anc = np.arange(_PANEL)
    for _ in range(4):
        anc = parent[anc]
        m = np.zeros((_PANEL, _PANEL), dtype=np.float32)
        m[anc, np.arange(_PANEL)] = 1.0
        maps.append(m)
    return np.stack(maps)


_ANC_MAPS = _ancestor_maps()


def _tc_body(x_ref, sel_ref, o_ref):
    xb = x_ref[...]
    head = xb[:, :_PANEL]
    m = head
    sel = sel_ref[...]
    for k in range(4):
        m = jnp.minimum(
            m, jax.lax.dot(head, sel[k], preferred_element_type=jnp.float32))
    fixed = jnp.concatenate([m, xb[:, _PANEL:]], axis=1)
    o_ref[...] = 1.0 / (1.0 + jnp.exp(-fixed))


def _tc_part(x):
    grid = (_BATCH - _SC_ROWS) // _TB
    return pl.pallas_call(
        _tc_body,
        grid=(grid,),
        in_specs=[
            pl.BlockSpec((_TB, _NODES), lambda i: (i, 0)),
            pl.BlockSpec((4, _PANEL, _PANEL), lambda i: (0, 0, 0)),
        ],
        out_specs=pl.BlockSpec((_TB, _NODES), lambda i: (i, 0)),
        out_shape=jax.ShapeDtypeStruct((_BATCH, _NODES), jnp.float32),
        compiler_params=pltpu.CompilerParams(
            dimension_semantics=("parallel",),
        ),
    )(x, jnp.asarray(_ANC_MAPS))


def _vgather(v, idx):
    return lax.gather(
        v,
        idx[:, None],
        dimension_numbers=lax.GatherDimensionNumbers(
            offset_dims=(), collapsed_slice_dims=(0,), start_index_map=(0,)),
        slice_sizes=(1,),
        mode=lax.GatherScatterMode.PROMISE_IN_BOUNDS,
    )


def _tree_fix(buf, r, p1, p2, p4, p_hi, i16):
    lo = buf[r, pl.ds(0, 16)]
    lo = jnp.minimum(lo, _vgather(lo, p1))
    lo = jnp.minimum(lo, _vgather(lo, p2))
    lo = jnp.minimum(lo, _vgather(lo, p4))
    buf[r, pl.ds(0, 16)] = lo
    hi = buf[r, pl.ds(16, 16)]
    gh = _vgather(lo, p_hi)
    hi = jnp.where(i16 < 15, jnp.minimum(hi, gh), hi)
    buf[r, pl.ds(16, 16)] = hi


def _compute_chunk(buf, p1, p2, p4, p_hi, i16):
    def row(r, c):
        _tree_fix(buf, r, p1, p2, p4, p_hi, i16)
        vals = [buf[r, pl.ds(j * 16, 16)] for j in range(64)]
        outs = [1.0 / (1.0 + jnp.exp(-v)) for v in vals]
        for j in range(64):
            buf[r, pl.ds(j * 16, 16)] = outs[j]
        return c

    lax.fori_loop(0, _CH, row, 0)


def _sc_body(x_hbm, o_hbm, b0, b1, ls0, ls1, ss0, ss1):
    wid = lax.axis_index("s") * _NC + lax.axis_index("c")
    r0 = wid * _ROWS_PER_W
    rx = _SC_BASE + r0

    i16 = lax.iota(jnp.int32, 16)
    par = lambda v: jnp.maximum((v - 1) >> 1, 0)
    p1 = par(i16)
    p2 = par(p1)
    p4 = par(par(p2))
    p_hi = (i16 + 15) >> 1

    bufs = (b0, b1)
    lsems = (ls0, ls1)
    ssems = (ss0, ss1)

    def ld(k):
        return pltpu.make_async_copy(
            x_hbm.at[pl.ds(rx + k * _CH, _CH), :], bufs[k % 2], lsems[k % 2])

    def st(k):
        return pltpu.make_async_copy(
            bufs[k % 2], o_hbm.at[pl.ds(r0 + k * _CH, _CH), :], ssems[k % 2])

    ld(0).start()
    for k in range(_NCHUNK):
        if k + 1 < _NCHUNK:
            if k - 1 >= 0:
                st(k - 1).wait()
            ld(k + 1).start()
        ld(k).wait()
        _compute_chunk(bufs[k % 2], p1, p2, p4, p_hi, i16)
        st(k).start()
    if _NCHUNK >= 2:
        st(_NCHUNK - 2).wait()
    st(_NCHUNK - 1).wait()


def _sc_part(x):
    mesh = plsc.VectorSubcoreMesh(core_axis_name="c", subcore_axis_name="s")
    run = functools.partial(
        pl.kernel,
        mesh=mesh,
        out_type=jax.ShapeDtypeStruct((_SC_ROWS, _NODES), jnp.float32),
        scratch_types=[
            pltpu.VMEM((_CH, _NODES), jnp.float32),
            pltpu.VMEM((_CH, _NODES), jnp.float32),
            pltpu.SemaphoreType.DMA,
            pltpu.SemaphoreType.DMA,
            pltpu.SemaphoreType.DMA,
            pltpu.SemaphoreType.DMA,
        ],
        compiler_params=pltpu.CompilerParams(use_tc_tiling_on_sc=True),
    )(_sc_body)
    return run(x)


@jax.jit
def kernel(x):
    sc_out = _sc_part(x)
    tc_out = _tc_part(x)
    return lax.dynamic_update_slice(tc_out, sc_out, (_SC_BASE, 0))


# speedup vs baseline: 1.1295x; 1.0013x over previous
import functools

import jax
import jax.numpy as jnp
import numpy as np
from jax import lax
from jax.experimental import pallas as pl
from jax.experimental.pallas import tpu as pltpu
from jax.experimental.pallas import tpu_sc as plsc

_BATCH = 16384
_NODES = 1024
_PANEL = 128

_SC_ROWS = 1024
_SC_BASE = _BATCH - _SC_ROWS
_TB = 2560
_INFO = plsc.get_sparse_core_info()
_NC, _NS = _INFO.num_cores, _INFO.num_subcores
_NW = _NC * _NS
_ROWS_PER_W = _SC_ROWS // _NW
_CH = 32
_NCHUNK = _ROWS_PER_W // _CH
_CHW = _CH * _NODES


def _ancestor_maps():
    parent = np.arange(_PANEL)
    parent[1:31] = (np.arange(1, 31) - 1) // 2
    maps = []
    anc = np.arange(_PANEL)
    for _ in range(4):
        anc = parent[anc]
        m = np.zeros((_PANEL, _PANEL), dtype=np.float32)
        m[anc, np.arange(_PANEL)] = 1.0
        maps.append(m)
    return np.stack(maps)


_ANC_MAPS = _ancestor_maps()


def _tc_body(x_ref, sel_ref, o_ref):
    xb = x_ref[...]
    head = xb[:, :_PANEL]
    m = head
    sel = sel_ref[...]
    for k in range(4):
        m = jnp.minimum(
            m, jax.lax.dot(head, sel[k], preferred_element_type=jnp.float32))
    fixed = jnp.concatenate([m, xb[:, _PANEL:]], axis=1)
    o_ref[...] = 1.0 / (1.0 + jnp.exp(-fixed))


def _tc_part(x):
    grid = (_BATCH - _SC_ROWS) // _TB
    return pl.pallas_call(
        _tc_body,
        grid=(grid,),
        in_specs=[
            pl.BlockSpec((_TB, _NODES), lambda i: (i, 0)),
            pl.BlockSpec((4, _PANEL, _PANEL), lambda i: (0, 0, 0)),
        ],
        out_specs=pl.BlockSpec((_TB, _NODES), lambda i: (i, 0)),
        out_shape=jax.ShapeDtypeStruct((_BATCH, _NODES), jnp.float32),
        compiler_params=pltpu.CompilerParams(
            dimension_semantics=("parallel",),
        ),
    )(x, jnp.asarray(_ANC_MAPS))


def _vgather(v, idx):
    return lax.gather(
        v,
        idx[:, None],
        dimension_numbers=lax.GatherDimensionNumbers(
            offset_dims=(), collapsed_slice_dims=(0,), start_index_map=(0,)),
        slice_sizes=(1,),
        mode=lax.GatherScatterMode.PROMISE_IN_BOUNDS,
    )


def _tree_fix(buf, r, p1, p2, p4, p_hi, i16):
    lo = buf[r, pl.ds(0, 16)]
    lo = jnp.minimum(lo, _vgather(lo, p1))
    lo = jnp.minimum(lo, _vgather(lo, p2))
    lo = jnp.minimum(lo, _vgather(lo, p4))
    buf[r, pl.ds(0, 16)] = lo
    hi = buf[r, pl.ds(16, 16)]
    gh = _vgather(lo, p_hi)
    hi = jnp.where(i16 < 15, jnp.minimum(hi, gh), hi)
    buf[r, pl.ds(16, 16)] = hi


def _compute_chunk(buf, p1, p2, p4, p_hi, i16):
    def row(r, c):
        _tree_fix(buf, r, p1, p2, p4, p_hi, i16)
        vals = [buf[r, pl.ds(j * 16, 16)] for j in range(64)]
        outs = [1.0 / (1.0 + jnp.exp(-v)) for v in vals]
        for j in range(64):
            buf[r, pl.ds(j * 16, 16)] = outs[j]
        return c

    lax.fori_loop(0, _CH, row, 0)


def _sc_body(x_hbm, o_hbm, b0, b1, ls0, ls1, ss0, ss1):
    wid = lax.axis_index("s") * _NC + lax.axis_index("c")
    r0 = wid * _ROWS_PER_W
    rx = _SC_BASE + r0

    i16 = lax.iota(jnp.int32, 16)
    par = lambda v: jnp.maximum((v - 1) >> 1, 0)
    p1 = par(i16)
    p2 = par(p1)
    p4 = par(par(p2))
    p_hi = (i16 + 15) >> 1

    bufs = (b0, b1)
    lsems = (ls0, ls1)
    ssems = (ss0, ss1)

    def ld(k):
        return pltpu.make_async_copy(
            x_hbm.at[pl.ds(rx + k * _CH, _CH), :], bufs[k % 2], lsems[k % 2])

    def st(k):
        return pltpu.make_async_copy(
            bufs[k % 2], o_hbm.at[pl.ds(r0 + k * _CH, _CH), :], ssems[k % 2])

    ld(0).start()
    for k in range(_NCHUNK):
        if k + 1 < _NCHUNK:
            if k - 1 >= 0:
                st(k - 1).wait()
            ld(k + 1).start()
        ld(k).wait()
        _compute_chunk(bufs[k % 2], p1, p2, p4, p_hi, i16)
        st(k).start()
    if _NCHUNK >= 2:
        st(_NCHUNK - 2).wait()
    st(_NCHUNK - 1).wait()


def _sc_part(x):
    mesh = plsc.VectorSubcoreMesh(core_axis_name="c", subcore_axis_name="s")
    run = functools.partial(
        pl.kernel,
        mesh=mesh,
        out_type=jax.ShapeDtypeStruct((_SC_ROWS, _NODES), jnp.float32),
        scratch_types=[
            pltpu.VMEM((_CH, _NODES), jnp.float32),
            pltpu.VMEM((_CH, _NODES), jnp.float32),
            pltpu.SemaphoreType.DMA,
            pltpu.SemaphoreType.DMA,
            pltpu.SemaphoreType.DMA,
            pltpu.SemaphoreType.DMA,
        ],
        compiler_params=pltpu.CompilerParams(use_tc_tiling_on_sc=True),
    )(_sc_body)
    return run(x)


@jax.jit
def kernel(x):
    sc_out = _sc_part(x)
    tc_out = _tc_part(x)
    return lax.dynamic_update_slice(tc_out, sc_out, (_SC_BASE, 0))
